# baseline (speedup 1.0000x reference)
import functools

import jax
import jax.numpy as jnp
from jax import lax
from jax.experimental import pallas as pl
from jax.experimental.pallas import tpu as pltpu
from jax.experimental.pallas import tpu_sc as plsc

B = 16384
D = 32
NC = 2
NS = 16
NW = NC * NS
BPW = B // NW
L = 16
IDX_CHUNK = 128
NCHUNK = BPW // IDX_CHUNK

_mesh = plsc.VectorSubcoreMesh(core_axis_name="c", subcore_axis_name="s")


@functools.partial(
    pl.kernel,
    mesh=_mesh,
    out_type=jax.ShapeDtypeStruct((B,), jnp.float32),
    compiler_params=pltpu.CompilerParams(
        needs_layout_passes=False, use_tc_tiling_on_sc=False
    ),
    scratch_types=[
        pltpu.VMEM((NCHUNK, IDX_CHUNK), jnp.int32),
        pltpu.VMEM((NCHUNK, IDX_CHUNK), jnp.int32),
        pltpu.VMEM((BPW, D), jnp.float32),
        pltpu.VMEM((BPW, D), jnp.float32),
        pltpu.VMEM((BPW,), jnp.float32),
        pltpu.SemaphoreType.DMA,
        pltpu.SemaphoreType.DMA,
    ],
)
def _sc_dot_kernel(t_idx_hbm, h_idx_hbm, t_tab_hbm, h_tab_hbm, out_hbm,
                   t_idx_v, h_idx_v, t_rows, h_rows, out_v, sem_t, sem_h):
    wid = lax.axis_index("s") * NC + lax.axis_index("c")
    base = wid * BPW

    pltpu.sync_copy(t_idx_hbm.at[wid], t_idx_v)
    pltpu.sync_copy(h_idx_hbm.at[wid], h_idx_v)

    for j in range(NCHUNK):
        pltpu.async_copy(
            t_tab_hbm.at[t_idx_v.at[j]],
            t_rows.at[pl.ds(j * IDX_CHUNK, IDX_CHUNK)],
            sem_t,
        )
        pltpu.async_copy(
            h_tab_hbm.at[h_idx_v.at[j]],
            h_rows.at[pl.ds(j * IDX_CHUNK, IDX_CHUNK)],
            sem_h,
        )
    for j in range(NCHUNK):
        pltpu.make_async_copy(
            t_tab_hbm.at[t_idx_v.at[j]],
            t_rows.at[pl.ds(j * IDX_CHUNK, IDX_CHUNK)],
            sem_t,
        ).wait()
        pltpu.make_async_copy(
            h_tab_hbm.at[h_idx_v.at[j]],
            h_rows.at[pl.ds(j * IDX_CHUNK, IDX_CHUNK)],
            sem_h,
        ).wait()

    lane = lax.iota(jnp.int32, L)

    def chunk_body(c, carry):
        row_ids = c * L + lane
        acc = jnp.zeros((L,), jnp.float32)
        for d in range(D):
            col = jnp.full((L,), d, jnp.int32)
            tv = plsc.load_gather(t_rows, [row_ids, col])
            hv = plsc.load_gather(h_rows, [row_ids, col])
            acc = acc + tv * hv
        out_v[pl.ds(c * L, L)] = acc
        return carry

    lax.fori_loop(0, BPW // L, chunk_body, 0)

    pltpu.sync_copy(out_v, out_hbm.at[pl.ds(base, BPW)])


def kernel(inputs, tumor_table, hospital_table):
    t_idx = inputs[:, 0].reshape(NW, NCHUNK, IDX_CHUNK)
    h_idx = inputs[:, 1].reshape(NW, NCHUNK, IDX_CHUNK)
    out = _sc_dot_kernel(t_idx, h_idx, tumor_table, hospital_table)
    return out[:, None]
